# baseline (speedup 1.0000x reference)
import functools

import jax
import jax.numpy as jnp
from jax import lax
from jax.experimental import pallas as pl
from jax.experimental.pallas import tpu as pltpu
from jax.experimental.pallas import tpu_sc as plsc

D = 768
E = 16
DFF = 4 * D
T = 2048
BT = 256
MAXB = 24
FC = 1024
NFF = DFF // FC
PADT = MAXB * BT
NC, NS = 2, 16
NW = NC * NS
RPW = T // NW


def _gate_body(x_ref, wg_ref, bg_ref, gate_ref, dest_ref, meta_ref):
    xf = x_ref[...]
    logits = jnp.dot(xf, wg_ref[...], preferred_element_type=jnp.float32)
    logits = logits + bg_ref[...]
    lane = lax.broadcasted_iota(jnp.int32, (T, E), 1)
    maxv = jnp.max(logits, axis=1, keepdims=True)
    idx = jnp.min(jnp.where(logits == maxv, lane, E), axis=1, keepdims=True)
    oh_i = (lane == idx).astype(jnp.int32)
    oh_f = oh_i.astype(jnp.float32)
    gate_ref[...] = logits + (oh_f - logits)

    c = oh_i
    k = 1
    while k < T:
        c = c + jnp.concatenate(
            [jnp.zeros((k, E), jnp.int32), c[: T - k, :]], axis=0)
        k *= 2
    counts = c[T - 1 : T, :]
    nb = (counts + (BT - 1)) // BT
    be = nb
    k = 1
    while k < E:
        be = be + jnp.concatenate(
            [jnp.zeros((1, k), jnp.int32), be[:, : E - k]], axis=1)
        k *= 2
    blk_off = be - nb
    pad_off = blk_off * BT
    total = be[:, E - 1 : E]

    biota = lax.broadcasted_iota(jnp.int32, (MAXB, E), 0)
    eob = jnp.sum((biota >= be).astype(jnp.int32), axis=1, keepdims=True)
    eob = jnp.minimum(eob, E - 1)
    meta_ref[...] = jnp.concatenate(
        [eob, total, jnp.zeros((7, 1), jnp.int32)], axis=0)

    dest_ref[...] = jnp.sum(oh_i * (pad_off + c - 1), axis=1, keepdims=True)


_gate_call = pl.pallas_call(
    _gate_body,
    out_shape=(
        jax.ShapeDtypeStruct((T, E), jnp.float32),
        jax.ShapeDtypeStruct((T, 1), jnp.int32),
        jax.ShapeDtypeStruct((MAXB + 8, 1), jnp.int32),
    ),
)


def _ffn_body(eob_ref, nb_ref, x_ref, wu_ref, bu_ref, wd_ref, bd_ref, o_ref):
    b = pl.program_id(0)
    j = pl.program_id(1)

    @pl.when(b < nb_ref[0])
    def _():
        xb = x_ref[...].astype(jnp.bfloat16)
        wu = wu_ref[0].astype(jnp.bfloat16)
        h = jnp.dot(xb, wu, preferred_element_type=jnp.float32) + bu_ref[0]
        h = jnp.maximum(h, 0.0).astype(jnp.bfloat16)
        wd = wd_ref[0].astype(jnp.bfloat16)
        part = jnp.dot(h, wd, preferred_element_type=jnp.float32)

        @pl.when(j == 0)
        def _():
            o_ref[...] = part + bd_ref[0]

        @pl.when(j > 0)
        def _():
            o_ref[...] += part


def _bclamp(b, nb):
    return jnp.minimum(b, nb[0] - 1)


def _wmap(b, j, eob, nb):
    bv = _bclamp(b, nb)
    return (eob[bv], 0, jnp.where(b < nb[0], j, NFF - 1))


def _wdmap(b, j, eob, nb):
    bv = _bclamp(b, nb)
    return (eob[bv], jnp.where(b < nb[0], j, NFF - 1), 0)


_ffn_grid_spec = pltpu.PrefetchScalarGridSpec(
    num_scalar_prefetch=2,
    grid=(MAXB, NFF),
    in_specs=[
        pl.BlockSpec((BT, D), lambda b, j, eob, nb: (_bclamp(b, nb), 0)),
        pl.BlockSpec((1, D, FC), _wmap),
        pl.BlockSpec((1, 1, FC), _wmap),
        pl.BlockSpec((1, FC, D), _wdmap),
        pl.BlockSpec((1, 1, D), lambda b, j, eob, nb: (eob[_bclamp(b, nb)], 0, 0)),
    ],
    out_specs=pl.BlockSpec((BT, D), lambda b, j, eob, nb: (_bclamp(b, nb), 0)),
)

_ffn_call = pl.pallas_call(
    _ffn_body,
    grid_spec=_ffn_grid_spec,
    out_shape=jax.ShapeDtypeStruct((PADT, D), jnp.float32),
)


def _sc_mesh():
    return plsc.VectorSubcoreMesh(
        core_axis_name="c", subcore_axis_name="s",
        num_cores=NC, num_subcores=NS)


def _scatter_body(xf_hbm, dest_hbm, out_hbm, idx_v, rows_v, sem):
    wid = lax.axis_index("s") * NC + lax.axis_index("c")
    base = wid * RPW
    pltpu.sync_copy(dest_hbm.at[pl.ds(base, RPW)], idx_v)
    pltpu.sync_copy(xf_hbm.at[pl.ds(base, RPW)], rows_v)
    pltpu.async_copy(rows_v, out_hbm.at[idx_v], sem).wait()


def _gather_body(ys_hbm, dest_hbm, out_hbm, idx_v, rows_v, sem):
    wid = lax.axis_index("s") * NC + lax.axis_index("c")
    base = wid * RPW
    pltpu.sync_copy(dest_hbm.at[pl.ds(base, RPW)], idx_v)
    pltpu.async_copy(ys_hbm.at[idx_v], rows_v, sem).wait()
    pltpu.sync_copy(rows_v, out_hbm.at[pl.ds(base, RPW)])


def _sc_scatter_call(xf, dest):
    fn = pl.kernel(
        _scatter_body,
        out_type=jax.ShapeDtypeStruct((PADT, D), jnp.float32),
        mesh=_sc_mesh(),
        scratch_types=[
            pltpu.VMEM((RPW,), jnp.int32),
            pltpu.VMEM((RPW, D), jnp.float32),
            pltpu.SemaphoreType.DMA,
        ],
    )
    return fn(xf, dest)


def _sc_gather_call(ys, dest):
    fn = pl.kernel(
        _gather_body,
        out_type=jax.ShapeDtypeStruct((T, D), jnp.float32),
        mesh=_sc_mesh(),
        scratch_types=[
            pltpu.VMEM((RPW,), jnp.int32),
            pltpu.VMEM((RPW, D), jnp.float32),
            pltpu.SemaphoreType.DMA,
        ],
    )
    return fn(ys, dest)


def kernel(x, W_gate, b_gate, W_up, b_up, W_down, b_down):
    Bx, Tx, C = x.shape
    xf = x.reshape(T, D)
    gate, dest2, meta = _gate_call(xf, W_gate, b_gate.reshape(1, E))
    dest = dest2.reshape(T)
    meta_f = meta.reshape(MAXB + 8)
    eob = meta_f[:MAXB]
    nb = meta_f[MAXB : MAXB + 1]
    xs = _sc_scatter_call(xf, dest)
    ys = _ffn_call(eob, nb, xs, W_up, b_up.reshape(E, 1, DFF),
                   W_down, b_down.reshape(E, 1, D))
    outf = _sc_gather_call(ys, dest)
    return outf.reshape(Bx, Tx, C), gate.reshape(Bx, Tx, E)


# speedup vs baseline: 1.0763x; 1.0763x over previous
import functools

import jax
import jax.numpy as jnp
from jax import lax
from jax.experimental import pallas as pl
from jax.experimental.pallas import tpu as pltpu
from jax.experimental.pallas import tpu_sc as plsc

D = 768
E = 16
DFF = 4 * D
T = 2048
BT = 256
MAXB = 24
FC = 3072
NFF = DFF // FC
PADT = MAXB * BT
NC, NS = 2, 16
NW = NC * NS
RPW = T // NW


def _gate_body(x_ref, wg_ref, bg_ref, gate_ref, dest_ref, meta_ref):
    xf = x_ref[...]
    logits = jnp.dot(xf, wg_ref[...], preferred_element_type=jnp.float32)
    logits = logits + bg_ref[...]
    lane = lax.broadcasted_iota(jnp.int32, (T, E), 1)
    maxv = jnp.max(logits, axis=1, keepdims=True)
    idx = jnp.min(jnp.where(logits == maxv, lane, E), axis=1, keepdims=True)
    oh_i = (lane == idx).astype(jnp.int32)
    oh_f = oh_i.astype(jnp.float32)
    gate_ref[...] = logits + (oh_f - logits)

    c = oh_i
    k = 1
    while k < T:
        c = c + jnp.concatenate(
            [jnp.zeros((k, E), jnp.int32), c[: T - k, :]], axis=0)
        k *= 2
    counts = c[T - 1 : T, :]
    nb = (counts + (BT - 1)) // BT
    be = nb
    k = 1
    while k < E:
        be = be + jnp.concatenate(
            [jnp.zeros((1, k), jnp.int32), be[:, : E - k]], axis=1)
        k *= 2
    blk_off = be - nb
    pad_off = blk_off * BT
    total = be[:, E - 1 : E]

    biota = lax.broadcasted_iota(jnp.int32, (MAXB, E), 0)
    eob = jnp.sum((biota >= be).astype(jnp.int32), axis=1, keepdims=True)
    eob = jnp.minimum(eob, E - 1)
    meta_ref[...] = jnp.concatenate(
        [eob, total, jnp.zeros((7, 1), jnp.int32)], axis=0)

    dest_ref[...] = jnp.sum(oh_i * (pad_off + c - 1), axis=1, keepdims=True)


_gate_call = pl.pallas_call(
    _gate_body,
    out_shape=(
        jax.ShapeDtypeStruct((T, E), jnp.float32),
        jax.ShapeDtypeStruct((T, 1), jnp.int32),
        jax.ShapeDtypeStruct((MAXB + 8, 1), jnp.int32),
    ),
)


def _ffn_body(eob_ref, nb_ref, x_ref, wu_ref, bu_ref, wd_ref, bd_ref, o_ref):
    b = pl.program_id(0)
    j = pl.program_id(1)

    @pl.when(b < nb_ref[0])
    def _():
        xb = x_ref[...].astype(jnp.bfloat16)
        wu = wu_ref[0].astype(jnp.bfloat16)
        h = jnp.dot(xb, wu, preferred_element_type=jnp.float32) + bu_ref[0]
        h = jnp.maximum(h, 0.0).astype(jnp.bfloat16)
        wd = wd_ref[0].astype(jnp.bfloat16)
        part = jnp.dot(h, wd, preferred_element_type=jnp.float32)

        @pl.when(j == 0)
        def _():
            o_ref[...] = part + bd_ref[0]

        @pl.when(j > 0)
        def _():
            o_ref[...] += part


def _bclamp(b, nb):
    return jnp.minimum(b, nb[0] - 1)


def _wmap(b, j, eob, nb):
    bv = _bclamp(b, nb)
    return (eob[bv], 0, jnp.where(b < nb[0], j, NFF - 1))


def _wdmap(b, j, eob, nb):
    bv = _bclamp(b, nb)
    return (eob[bv], jnp.where(b < nb[0], j, NFF - 1), 0)


_ffn_grid_spec = pltpu.PrefetchScalarGridSpec(
    num_scalar_prefetch=2,
    grid=(MAXB, NFF),
    in_specs=[
        pl.BlockSpec((BT, D), lambda b, j, eob, nb: (_bclamp(b, nb), 0)),
        pl.BlockSpec((1, D, FC), _wmap),
        pl.BlockSpec((1, 1, FC), _wmap),
        pl.BlockSpec((1, FC, D), _wdmap),
        pl.BlockSpec((1, 1, D), lambda b, j, eob, nb: (eob[_bclamp(b, nb)], 0, 0)),
    ],
    out_specs=pl.BlockSpec((BT, D), lambda b, j, eob, nb: (_bclamp(b, nb), 0)),
)

_ffn_call = pl.pallas_call(
    _ffn_body,
    grid_spec=_ffn_grid_spec,
    out_shape=jax.ShapeDtypeStruct((PADT, D), jnp.float32),
)


def _sc_mesh():
    return plsc.VectorSubcoreMesh(
        core_axis_name="c", subcore_axis_name="s",
        num_cores=NC, num_subcores=NS)


def _scatter_body(xf_hbm, dest_hbm, out_hbm, idx_v, rows_v, sem):
    wid = lax.axis_index("s") * NC + lax.axis_index("c")
    base = wid * RPW
    pltpu.sync_copy(dest_hbm.at[pl.ds(base, RPW)], idx_v)
    pltpu.sync_copy(xf_hbm.at[pl.ds(base, RPW)], rows_v)
    pltpu.async_copy(rows_v, out_hbm.at[idx_v], sem).wait()


def _gather_body(ys_hbm, dest_hbm, out_hbm, idx_v, rows_v, sem):
    wid = lax.axis_index("s") * NC + lax.axis_index("c")
    base = wid * RPW
    pltpu.sync_copy(dest_hbm.at[pl.ds(base, RPW)], idx_v)
    pltpu.async_copy(ys_hbm.at[idx_v], rows_v, sem).wait()
    pltpu.sync_copy(rows_v, out_hbm.at[pl.ds(base, RPW)])


def _sc_scatter_call(xf, dest):
    fn = pl.kernel(
        _scatter_body,
        out_type=jax.ShapeDtypeStruct((PADT, D), jnp.float32),
        mesh=_sc_mesh(),
        scratch_types=[
            pltpu.VMEM((RPW,), jnp.int32),
            pltpu.VMEM((RPW, D), jnp.float32),
            pltpu.SemaphoreType.DMA,
        ],
    )
    return fn(xf, dest)


def _sc_gather_call(ys, dest):
    fn = pl.kernel(
        _gather_body,
        out_type=jax.ShapeDtypeStruct((T, D), jnp.float32),
        mesh=_sc_mesh(),
        scratch_types=[
            pltpu.VMEM((RPW,), jnp.int32),
            pltpu.VMEM((RPW, D), jnp.float32),
            pltpu.SemaphoreType.DMA,
        ],
    )
    return fn(ys, dest)


def kernel(x, W_gate, b_gate, W_up, b_up, W_down, b_down):
    Bx, Tx, C = x.shape
    xf = x.reshape(T, D)
    gate, dest2, meta = _gate_call(xf, W_gate, b_gate.reshape(1, E))
    dest = dest2.reshape(T)
    meta_f = meta.reshape(MAXB + 8)
    eob = meta_f[:MAXB]
    nb = meta_f[MAXB : MAXB + 1]
    xs = _sc_scatter_call(xf, dest)
    ys = _ffn_call(eob, nb, xs, W_up, b_up.reshape(E, 1, DFF),
                   W_down, b_down.reshape(E, 1, D))
    outf = _sc_gather_call(ys, dest)
    return outf.reshape(Bx, Tx, C), gate.reshape(Bx, Tx, E)
